# Initial kernel scaffold; baseline (speedup 1.0000x reference)
import jax
import jax.numpy as jnp
from jax.experimental import pallas as pl


def kernel(x, edge_index, batch, shape_emb, color_emb, W1, b1, W2, b2, Wlin, blin):
    raise NotImplementedError("write your pallas kernel here")


# speedup vs baseline: 10.3422x; 10.3422x over previous
import functools

import jax
import jax.numpy as jnp
from jax import lax
from jax.experimental import pallas as pl
from jax.experimental.pallas import tpu as pltpu
from jax.experimental.pallas import tpu_sc as plsc

N = 100000
E = 1600000
F = 32
N_SHAPE = 16
N_COLOR = 8
N_GRAPHS = 256

NC = 2
NS = 16
H = N // NC
CHUNK = 128
NCH = E // CHUNK
SC_ITERS = (NCH + NS - 1) // NS
JUNK = 512
WB = 1000
NWB = H // WB
WB_ITERS = (NWB + NS - 1) // NS
WB2 = 500
NWB2 = H // WB2
WB2_ITERS = (NWB2 + NS - 1) // NS

_MESH = plsc.VectorSubcoreMesh(
    core_axis_name="c", subcore_axis_name="s", num_cores=NC, num_subcores=NS
)

BLK = 2000
NB = N // BLK

_PREC = lax.Precision.HIGHEST


@functools.partial(
    pl.kernel,
    out_type=jax.ShapeDtypeStruct((N,), jnp.float32),
    mesh=_MESH,
    compiler_params=pltpu.CompilerParams(use_tc_tiling_on_sc=False),
    scratch_types=[
        pltpu.VMEM_SHARED((H + JUNK,), jnp.float32),
        pltpu.VMEM((CHUNK,), jnp.int32),
        pltpu.VMEM((CHUNK,), jnp.int32),
        pltpu.VMEM((CHUNK,), jnp.float32),
        pltpu.VMEM((WB,), jnp.float32),
        pltpu.SemaphoreType.DMA,
    ],
)
def _deg_sc(dst_hbm, ones_hbm, zeros_hbm, deg_hbm, acc, dbuf, ibuf, obuf, wbuf, sem):
    c = lax.axis_index("c")
    s = lax.axis_index("s")
    lo = c * H

    pltpu.sync_copy(ones_hbm, obuf)
    pltpu.sync_copy(zeros_hbm, wbuf)

    def zinit(i, carry):
        j = s + NS * i

        @pl.when(j < NWB)
        def _():
            pltpu.sync_copy(wbuf, acc.at[pl.ds(j * WB, WB)])

        return carry

    lax.fori_loop(0, WB_ITERS, zinit, 0)
    plsc.subcore_barrier()

    def body(i, carry):
        j = s + NS * i

        @pl.when(j < NCH)
        def _():
            base = j * CHUNK
            pltpu.sync_copy(dst_hbm.at[pl.ds(base, CHUNK)], dbuf)
            for k in range(CHUNK // 16):
                d = dbuf[pl.ds(k * 16, 16)]
                owned = (d >= lo) & (d < lo + H)
                junk = H + (d & (JUNK - 1))
                ibuf[pl.ds(k * 16, 16)] = jnp.where(owned, d - lo, junk)
            pltpu.sync_copy(obuf, acc.at[ibuf], add=True)

        return carry

    lax.fori_loop(0, SC_ITERS, body, 0)
    plsc.subcore_barrier()

    def wback(i, carry):
        j = s + NS * i

        @pl.when(j < NWB)
        def _():
            pltpu.sync_copy(acc.at[pl.ds(j * WB, WB)], wbuf)
            pltpu.sync_copy(wbuf, deg_hbm.at[pl.ds(lo + j * WB, WB)])

        return carry

    lax.fori_loop(0, WB_ITERS, wback, 0)


@functools.partial(
    pl.kernel,
    out_type=jax.ShapeDtypeStruct((N, F), jnp.float32),
    mesh=_MESH,
    compiler_params=pltpu.CompilerParams(use_tc_tiling_on_sc=False),
    scratch_types=[
        pltpu.VMEM_SHARED((H + JUNK, F), jnp.float32),
        pltpu.VMEM((CHUNK,), jnp.int32),
        pltpu.VMEM((CHUNK,), jnp.int32),
        pltpu.VMEM((CHUNK,), jnp.int32),
        pltpu.VMEM((CHUNK, F), jnp.float32),
        pltpu.VMEM((WB2, F), jnp.float32),
        pltpu.SemaphoreType.DMA,
    ],
)
def _agg_sc(g_hbm, src_hbm, dst_hbm, zeros_hbm, out_hbm, acc, sbuf, dbuf, ibuf, rbuf, wbuf, sem):
    c = lax.axis_index("c")
    s = lax.axis_index("s")
    lo = c * H

    pltpu.sync_copy(zeros_hbm, wbuf)

    def zinit(i, carry):
        j = s + NS * i

        @pl.when(j < NWB2)
        def _():
            pltpu.sync_copy(wbuf, acc.at[pl.ds(j * WB2, WB2)])

        return carry

    lax.fori_loop(0, WB2_ITERS, zinit, 0)
    plsc.subcore_barrier()

    def body(i, carry):
        j = s + NS * i

        @pl.when(j < NCH)
        def _():
            base = j * CHUNK
            pltpu.sync_copy(src_hbm.at[pl.ds(base, CHUNK)], sbuf)
            pltpu.sync_copy(dst_hbm.at[pl.ds(base, CHUNK)], dbuf)
            pltpu.async_copy(g_hbm.at[sbuf], rbuf, sem).wait()
            for k in range(CHUNK // 16):
                d = dbuf[pl.ds(k * 16, 16)]
                owned = (d >= lo) & (d < lo + H)
                junk = H + (d & (JUNK - 1))
                ibuf[pl.ds(k * 16, 16)] = jnp.where(owned, d - lo, junk)
            pltpu.sync_copy(rbuf, acc.at[ibuf], add=True)

        return carry

    lax.fori_loop(0, SC_ITERS, body, 0)
    plsc.subcore_barrier()

    def wback(i, carry):
        j = s + NS * i

        @pl.when(j < NWB2)
        def _():
            pltpu.sync_copy(acc.at[pl.ds(j * WB2, WB2)], wbuf)
            pltpu.sync_copy(wbuf, out_hbm.at[pl.ds(lo + j * WB2, WB2)])

        return carry

    lax.fori_loop(0, WB2_ITERS, wback, 0)


def _prep_body(x0_ref, x1_ref, deg_ref, se_ref, ce_ref, g0_ref):
    oh_s = (x0_ref[...] == lax.broadcasted_iota(jnp.int32, (BLK, N_SHAPE), 1))
    oh_c = (x1_ref[...] == lax.broadcasted_iota(jnp.int32, (BLK, N_COLOR), 1))
    h0 = jnp.dot(oh_s.astype(jnp.float32), se_ref[...], precision=_PREC)
    h0 = h0 + jnp.dot(oh_c.astype(jnp.float32), ce_ref[...], precision=_PREC)
    dinv = lax.rsqrt(deg_ref[...] + 1.0)
    g0_ref[...] = h0 * dinv


def _layer_body(s_ref, g_ref, deg_ref, w_ref, b_ref, gout_ref):
    dinv = lax.rsqrt(deg_ref[...] + 1.0)
    z = (s_ref[...] + g_ref[...]) * dinv
    h = jnp.maximum(jnp.dot(z, w_ref[...], precision=_PREC) + b_ref[...], 0.0)
    gout_ref[...] = h * dinv


def _pool_body(s_ref, g_ref, deg_ref, w_ref, b_ref, batch_ref, sums_ref, cnt_ref):
    i = pl.program_id(0)
    dinv = lax.rsqrt(deg_ref[...] + 1.0)
    z = (s_ref[...] + g_ref[...]) * dinv
    h = jnp.maximum(jnp.dot(z, w_ref[...], precision=_PREC) + b_ref[...], 0.0)
    oh = (batch_ref[...] == lax.broadcasted_iota(jnp.int32, (BLK, N_GRAPHS), 1))
    oh = oh.astype(jnp.float32)
    ps = lax.dot_general(oh, h, (((0,), (0,)), ((), ())), precision=_PREC)
    pc = jnp.sum(oh, axis=0)[:, None]

    @pl.when(i == 0)
    def _():
        sums_ref[...] = ps
        cnt_ref[...] = pc

    @pl.when(i != 0)
    def _():
        sums_ref[...] += ps
        cnt_ref[...] += pc


def _head_body(sums_ref, cnt_ref, wl_ref, bl_ref, out_ref):
    hg = sums_ref[...] / jnp.maximum(cnt_ref[...], 1.0)
    out_ref[...] = jnp.dot(hg, wl_ref[...], precision=_PREC) + bl_ref[...]


def _row_spec(width):
    return pl.BlockSpec((BLK, width), lambda i: (i, 0))


def _full_spec(shape):
    return pl.BlockSpec(shape, lambda i: (0, 0))


def kernel(x, edge_index, batch, shape_emb, color_emb, W1, b1, W2, b2, Wlin, blin):
    x0 = x[:, 0:1]
    x1 = x[:, 1:2]
    src = edge_index[0]
    dst = edge_index[1]

    ones128 = jnp.ones((CHUNK,), jnp.float32)
    zeros1d = jnp.zeros((WB,), jnp.float32)
    zeros2d = jnp.zeros((WB2, F), jnp.float32)

    deg = _deg_sc(dst, ones128, zeros1d)
    deg2 = deg[:, None]

    g0 = pl.pallas_call(
        _prep_body,
        grid=(NB,),
        in_specs=[
            _row_spec(1), _row_spec(1), _row_spec(1),
            _full_spec((N_SHAPE, F)), _full_spec((N_COLOR, F)),
        ],
        out_specs=_row_spec(F),
        out_shape=jax.ShapeDtypeStruct((N, F), jnp.float32),
    )(x0, x1, deg2, shape_emb, color_emb)

    s1 = _agg_sc(g0, src, dst, zeros2d)

    g1 = pl.pallas_call(
        _layer_body,
        grid=(NB,),
        in_specs=[
            _row_spec(F), _row_spec(F), _row_spec(1),
            _full_spec((F, F)), _full_spec((1, F)),
        ],
        out_specs=_row_spec(F),
        out_shape=jax.ShapeDtypeStruct((N, F), jnp.float32),
    )(s1, g0, deg2, W1, b1[None, :])

    s2 = _agg_sc(g1, src, dst, zeros2d)

    sums, cnt = pl.pallas_call(
        _pool_body,
        grid=(NB,),
        in_specs=[
            _row_spec(F), _row_spec(F), _row_spec(1),
            _full_spec((F, F)), _full_spec((1, F)), _row_spec(1),
        ],
        out_specs=[
            _full_spec((N_GRAPHS, F)),
            _full_spec((N_GRAPHS, 1)),
        ],
        out_shape=[
            jax.ShapeDtypeStruct((N_GRAPHS, F), jnp.float32),
            jax.ShapeDtypeStruct((N_GRAPHS, 1), jnp.float32),
        ],
    )(s2, g1, deg2, W2, b2[None, :], batch[:, None])

    out = pl.pallas_call(
        _head_body,
        grid=(1,),
        in_specs=[
            _full_spec((N_GRAPHS, F)),
            _full_spec((N_GRAPHS, 1)),
            _full_spec((F, blin.shape[0])),
            _full_spec((1, blin.shape[0])),
        ],
        out_specs=_full_spec((N_GRAPHS, blin.shape[0])),
        out_shape=jax.ShapeDtypeStruct((N_GRAPHS, blin.shape[0]), jnp.float32),
    )(sums, cnt, Wlin, blin[None, :])

    return out


# speedup vs baseline: 20.9320x; 2.0239x over previous
import functools

import jax
import jax.numpy as jnp
from jax import lax
from jax.experimental import pallas as pl
from jax.experimental.pallas import tpu as pltpu
from jax.experimental.pallas import tpu_sc as plsc

N = 100000
E = 1600000
F = 32
N_SHAPE = 16
N_COLOR = 8
N_GRAPHS = 256

NC = 2
NS = 16
H = N // NC
CHUNK = 128
K = 4
KD = 16
BLK_E = CHUNK * KD * NS
E_PAD = ((E + BLK_E - 1) // BLK_E) * BLK_E
NCHP = E_PAD // CHUNK
NBLK = NCHP // K
SUB_ITERS = NBLK // NS
NBLKD = NCHP // KD
SUB_ITERS_D = NBLKD // NS
JUNK = 512
WB = 1000
NWB = H // WB
WB_ITERS = (NWB + NS - 1) // NS
WB2 = 125
NWB2 = H // WB2
WB2_ITERS = (NWB2 + NS - 1) // NS

_MESH = plsc.VectorSubcoreMesh(
    core_axis_name="c", subcore_axis_name="s", num_cores=NC, num_subcores=NS
)

BLK = 2000
NB = N // BLK
LBLK = 128
NLB = NCHP // LBLK

_PREC = lax.Precision.HIGHEST


def _ldst_body(dst_ref, l0_ref, l1_ref):
    d = dst_ref[...]
    junk = H + (d & (JUNK - 1))
    l0_ref[...] = jnp.where(d < H, d, junk)
    l1_ref[...] = jnp.where((d >= H) & (d < N), d - H, junk)


@functools.partial(
    pl.kernel,
    out_type=jax.ShapeDtypeStruct((N,), jnp.float32),
    mesh=_MESH,
    compiler_params=pltpu.CompilerParams(use_tc_tiling_on_sc=False),
    scratch_types=[
        pltpu.VMEM_SHARED((H + JUNK,), jnp.float32),
        pltpu.VMEM((KD, CHUNK), jnp.int32),
        pltpu.VMEM((CHUNK,), jnp.float32),
        pltpu.VMEM((WB,), jnp.float32),
        pltpu.SemaphoreType.DMA,
    ],
)
def _deg_sc(l0_hbm, l1_hbm, ones_hbm, zeros_hbm, deg_hbm, acc, ibuf, obuf, wbuf, sem):
    c = lax.axis_index("c")
    s = lax.axis_index("s")
    lo = c * H

    pltpu.sync_copy(ones_hbm, obuf)
    pltpu.sync_copy(zeros_hbm, wbuf)

    def zinit(i, carry):
        j = s + NS * i

        @pl.when(j < NWB)
        def _():
            pltpu.sync_copy(wbuf, acc.at[pl.ds(j * WB, WB)])

        return carry

    lax.fori_loop(0, WB_ITERS, zinit, 0)
    plsc.subcore_barrier()

    def body(i, carry):
        b = s + NS * i
        row = b * KD

        @pl.when(c == 0)
        def _():
            pltpu.sync_copy(l0_hbm.at[pl.ds(row, KD)], ibuf)

        @pl.when(c == 1)
        def _():
            pltpu.sync_copy(l1_hbm.at[pl.ds(row, KD)], ibuf)

        for k in range(KD):
            pltpu.sync_copy(obuf, acc.at[ibuf.at[k]], add=True)

        return carry

    lax.fori_loop(0, SUB_ITERS_D, body, 0)
    plsc.subcore_barrier()

    def wback(i, carry):
        j = s + NS * i

        @pl.when(j < NWB)
        def _():
            pltpu.sync_copy(acc.at[pl.ds(j * WB, WB)], wbuf)
            pltpu.sync_copy(wbuf, deg_hbm.at[pl.ds(lo + j * WB, WB)])

        return carry

    lax.fori_loop(0, WB_ITERS, wback, 0)


@functools.partial(
    pl.kernel,
    out_type=jax.ShapeDtypeStruct((N, F), jnp.float32),
    mesh=_MESH,
    compiler_params=pltpu.CompilerParams(use_tc_tiling_on_sc=False),
    scratch_types=[
        pltpu.VMEM_SHARED((H + JUNK, F), jnp.float32),
        pltpu.VMEM((K, CHUNK), jnp.int32),
        pltpu.VMEM((K, CHUNK), jnp.int32),
        pltpu.VMEM((K, CHUNK, F), jnp.float32),
        pltpu.VMEM((WB2, F), jnp.float32),
        pltpu.SemaphoreType.DMA,
        pltpu.SemaphoreType.DMA,
    ],
)
def _agg_sc(g_hbm, src_hbm, l0_hbm, l1_hbm, zeros_hbm, out_hbm,
            acc, sbuf, ibuf, rbuf, wbuf, gsem, ssem):
    c = lax.axis_index("c")
    s = lax.axis_index("s")
    lo = c * H

    pltpu.sync_copy(zeros_hbm, wbuf)

    def zinit(i, carry):
        j = s + NS * i

        @pl.when(j < NWB2)
        def _():
            pltpu.sync_copy(wbuf, acc.at[pl.ds(j * WB2, WB2)])

        return carry

    lax.fori_loop(0, WB2_ITERS, zinit, 0)
    plsc.subcore_barrier()

    def body(i, carry):
        b = s + NS * i
        row = b * K

        pltpu.sync_copy(src_hbm.at[pl.ds(row, K)], sbuf)

        @pl.when(c == 0)
        def _():
            pltpu.sync_copy(l0_hbm.at[pl.ds(row, K)], ibuf)

        @pl.when(c == 1)
        def _():
            pltpu.sync_copy(l1_hbm.at[pl.ds(row, K)], ibuf)

        gh = [
            pltpu.async_copy(g_hbm.at[sbuf.at[k]], rbuf.at[k], gsem)
            for k in range(K)
        ]
        sh = []
        for k in range(K):
            gh[k].wait()
            sh.append(
                pltpu.async_copy(rbuf.at[k], acc.at[ibuf.at[k]], ssem, add=True)
            )
        for h in sh:
            h.wait()

        return carry

    lax.fori_loop(0, SUB_ITERS, body, 0)
    plsc.subcore_barrier()

    def wback(i, carry):
        j = s + NS * i

        @pl.when(j < NWB2)
        def _():
            pltpu.sync_copy(acc.at[pl.ds(j * WB2, WB2)], wbuf)
            pltpu.sync_copy(wbuf, out_hbm.at[pl.ds(lo + j * WB2, WB2)])

        return carry

    lax.fori_loop(0, WB2_ITERS, wback, 0)


def _prep_body(x0_ref, x1_ref, deg_ref, se_ref, ce_ref, g0_ref):
    oh_s = (x0_ref[...] == lax.broadcasted_iota(jnp.int32, (BLK, N_SHAPE), 1))
    oh_c = (x1_ref[...] == lax.broadcasted_iota(jnp.int32, (BLK, N_COLOR), 1))
    h0 = jnp.dot(oh_s.astype(jnp.float32), se_ref[...], precision=_PREC)
    h0 = h0 + jnp.dot(oh_c.astype(jnp.float32), ce_ref[...], precision=_PREC)
    dinv = lax.rsqrt(deg_ref[...] + 1.0)
    g0_ref[...] = h0 * dinv


def _layer_body(s_ref, g_ref, deg_ref, w_ref, b_ref, gout_ref):
    dinv = lax.rsqrt(deg_ref[...] + 1.0)
    z = (s_ref[...] + g_ref[...]) * dinv
    h = jnp.maximum(jnp.dot(z, w_ref[...], precision=_PREC) + b_ref[...], 0.0)
    gout_ref[...] = h * dinv


def _pool_body(s_ref, g_ref, deg_ref, w_ref, b_ref, batch_ref, sums_ref, cnt_ref):
    i = pl.program_id(0)
    dinv = lax.rsqrt(deg_ref[...] + 1.0)
    z = (s_ref[...] + g_ref[...]) * dinv
    h = jnp.maximum(jnp.dot(z, w_ref[...], precision=_PREC) + b_ref[...], 0.0)
    oh = (batch_ref[...] == lax.broadcasted_iota(jnp.int32, (BLK, N_GRAPHS), 1))
    oh = oh.astype(jnp.float32)
    ps = lax.dot_general(oh, h, (((0,), (0,)), ((), ())), precision=_PREC)
    pc = jnp.sum(oh, axis=0)[:, None]

    @pl.when(i == 0)
    def _():
        sums_ref[...] = ps
        cnt_ref[...] = pc

    @pl.when(i != 0)
    def _():
        sums_ref[...] += ps
        cnt_ref[...] += pc


def _head_body(sums_ref, cnt_ref, wl_ref, bl_ref, out_ref):
    hg = sums_ref[...] / jnp.maximum(cnt_ref[...], 1.0)
    out_ref[...] = jnp.dot(hg, wl_ref[...], precision=_PREC) + bl_ref[...]


def _row_spec(width):
    return pl.BlockSpec((BLK, width), lambda i: (i, 0))


def _full_spec(shape):
    return pl.BlockSpec(shape, lambda i: (0, 0))


def kernel(x, edge_index, batch, shape_emb, color_emb, W1, b1, W2, b2, Wlin, blin):
    x0 = x[:, 0:1]
    x1 = x[:, 1:2]
    src = edge_index[0]
    dst = edge_index[1]

    pad = E_PAD - E
    src_p = jnp.concatenate([src, jnp.zeros((pad,), src.dtype)]).reshape(NCHP, CHUNK)
    dst_p = jnp.concatenate([dst, jnp.full((pad,), N, dst.dtype)]).reshape(NCHP, CHUNK)

    ldst0, ldst1 = pl.pallas_call(
        _ldst_body,
        grid=(NLB,),
        in_specs=[pl.BlockSpec((LBLK, CHUNK), lambda i: (i, 0))],
        out_specs=[
            pl.BlockSpec((LBLK, CHUNK), lambda i: (i, 0)),
            pl.BlockSpec((LBLK, CHUNK), lambda i: (i, 0)),
        ],
        out_shape=[
            jax.ShapeDtypeStruct((NCHP, CHUNK), jnp.int32),
            jax.ShapeDtypeStruct((NCHP, CHUNK), jnp.int32),
        ],
    )(dst_p.astype(jnp.int32))

    src_p = src_p.astype(jnp.int32)
    ones128 = jnp.ones((CHUNK,), jnp.float32)
    zeros1d = jnp.zeros((WB,), jnp.float32)
    zeros2d = jnp.zeros((WB2, F), jnp.float32)

    deg = _deg_sc(ldst0, ldst1, ones128, zeros1d)
    deg2 = deg[:, None]

    g0 = pl.pallas_call(
        _prep_body,
        grid=(NB,),
        in_specs=[
            _row_spec(1), _row_spec(1), _row_spec(1),
            _full_spec((N_SHAPE, F)), _full_spec((N_COLOR, F)),
        ],
        out_specs=_row_spec(F),
        out_shape=jax.ShapeDtypeStruct((N, F), jnp.float32),
    )(x0, x1, deg2, shape_emb, color_emb)

    s1 = _agg_sc(g0, src_p, ldst0, ldst1, zeros2d)

    g1 = pl.pallas_call(
        _layer_body,
        grid=(NB,),
        in_specs=[
            _row_spec(F), _row_spec(F), _row_spec(1),
            _full_spec((F, F)), _full_spec((1, F)),
        ],
        out_specs=_row_spec(F),
        out_shape=jax.ShapeDtypeStruct((N, F), jnp.float32),
    )(s1, g0, deg2, W1, b1[None, :])

    s2 = _agg_sc(g1, src_p, ldst0, ldst1, zeros2d)

    sums, cnt = pl.pallas_call(
        _pool_body,
        grid=(NB,),
        in_specs=[
            _row_spec(F), _row_spec(F), _row_spec(1),
            _full_spec((F, F)), _full_spec((1, F)), _row_spec(1),
        ],
        out_specs=[
            _full_spec((N_GRAPHS, F)),
            _full_spec((N_GRAPHS, 1)),
        ],
        out_shape=[
            jax.ShapeDtypeStruct((N_GRAPHS, F), jnp.float32),
            jax.ShapeDtypeStruct((N_GRAPHS, 1), jnp.float32),
        ],
    )(s2, g1, deg2, W2, b2[None, :], batch[:, None])

    out = pl.pallas_call(
        _head_body,
        grid=(1,),
        in_specs=[
            _full_spec((N_GRAPHS, F)),
            _full_spec((N_GRAPHS, 1)),
            _full_spec((F, blin.shape[0])),
            _full_spec((1, blin.shape[0])),
        ],
        out_specs=_full_spec((N_GRAPHS, blin.shape[0])),
        out_shape=jax.ShapeDtypeStruct((N_GRAPHS, blin.shape[0]), jnp.float32),
    )(sums, cnt, Wlin, blin[None, :])

    return out


# speedup vs baseline: 25.5323x; 1.2198x over previous
import functools

import jax
import jax.numpy as jnp
from jax import lax
from jax.experimental import pallas as pl
from jax.experimental.pallas import tpu as pltpu
from jax.experimental.pallas import tpu_sc as plsc

N = 100000
E = 1600000
F = 32
FH = F // 2
N_SHAPE = 16
N_COLOR = 8
N_GRAPHS = 256

NC = 2
NS = 16
CHUNK = 128
K = 8
KD = 8
BLK_E = CHUNK * 2 * KD * NS
E_PAD = ((E + BLK_E - 1) // BLK_E) * BLK_E
NCHP = E_PAD // CHUNK
NBLK = NCHP // K
SUB_ITERS = NBLK // NS
NBLKD_C = NCHP // 2 // KD
SUB_ITERS_D = NBLKD_C // NS
JUNK = 500
JUNKD = 512
ACC_D = N + JUNKD
ACC_A = N + JUNK
WB = 1000
NWB = N // WB
WB_ITERS = (NWB + NS - 1) // NS
WB2 = 125
NWB2_I = ACC_A // WB2
NWB2_O = N // WB2
WB2_ITERS = (NWB2_I + NS - 1) // NS

_MESH = plsc.VectorSubcoreMesh(
    core_axis_name="c", subcore_axis_name="s", num_cores=NC, num_subcores=NS
)

BLK = 2000
NB = N // BLK

_PREC = lax.Precision.HIGHEST


@functools.partial(
    pl.kernel,
    out_type=[
        jax.ShapeDtypeStruct((N,), jnp.float32),
        jax.ShapeDtypeStruct((N,), jnp.float32),
    ],
    mesh=_MESH,
    compiler_params=pltpu.CompilerParams(use_tc_tiling_on_sc=False),
    scratch_types=[
        pltpu.VMEM_SHARED((ACC_D,), jnp.float32),
        pltpu.VMEM((KD, CHUNK), jnp.int32),
        pltpu.VMEM((CHUNK,), jnp.float32),
        pltpu.VMEM((WB,), jnp.float32),
        pltpu.SemaphoreType.DMA,
    ],
)
def _deg_sc(dst_hbm, ones_hbm, zeros_hbm, degA_hbm, degB_hbm,
            acc, ibuf, obuf, wbuf, sem):
    c = lax.axis_index("c")
    s = lax.axis_index("s")

    pltpu.sync_copy(ones_hbm, obuf)
    pltpu.sync_copy(zeros_hbm, wbuf)

    def zinit(i, carry):
        j = s + NS * i

        @pl.when(j < NWB)
        def _():
            pltpu.sync_copy(wbuf, acc.at[pl.ds(j * WB, WB)])

        return carry

    lax.fori_loop(0, WB_ITERS, zinit, 0)

    @pl.when(s == 0)
    def _():
        pltpu.sync_copy(wbuf.at[pl.ds(0, JUNKD)], acc.at[pl.ds(N, JUNKD)])

    plsc.subcore_barrier()

    def body(i, carry):
        b = c * NBLKD_C + s + NS * i
        row = b * KD
        pltpu.sync_copy(dst_hbm.at[pl.ds(row, KD)], ibuf)
        for k in range(KD):
            pltpu.sync_copy(obuf, acc.at[ibuf.at[k]], add=True)
        return carry

    lax.fori_loop(0, SUB_ITERS_D, body, 0)
    plsc.subcore_barrier()

    def wback(i, carry):
        j = s + NS * i

        @pl.when(j < NWB)
        def _():
            pltpu.sync_copy(acc.at[pl.ds(j * WB, WB)], wbuf)

            @pl.when(c == 0)
            def _():
                pltpu.sync_copy(wbuf, degA_hbm.at[pl.ds(j * WB, WB)])

            @pl.when(c == 1)
            def _():
                pltpu.sync_copy(wbuf, degB_hbm.at[pl.ds(j * WB, WB)])

        return carry

    lax.fori_loop(0, WB_ITERS, wback, 0)


@functools.partial(
    pl.kernel,
    out_type=[
        jax.ShapeDtypeStruct((N, FH), jnp.float32),
        jax.ShapeDtypeStruct((N, FH), jnp.float32),
    ],
    mesh=_MESH,
    compiler_params=pltpu.CompilerParams(use_tc_tiling_on_sc=False),
    scratch_types=[
        pltpu.VMEM_SHARED((ACC_A, FH), jnp.float32),
        pltpu.VMEM((K, CHUNK), jnp.int32),
        pltpu.VMEM((K, CHUNK), jnp.int32),
        pltpu.VMEM((K, CHUNK, FH), jnp.float32),
        pltpu.VMEM((WB2, FH), jnp.float32),
        pltpu.SemaphoreType.DMA,
        pltpu.SemaphoreType.DMA,
    ],
)
def _agg_sc(gL_hbm, gR_hbm, src_hbm, dst_hbm, zeros_hbm, outL_hbm, outR_hbm,
            acc, sbuf, ibuf, rbuf, wbuf, gsem, ssem):
    c = lax.axis_index("c")
    s = lax.axis_index("s")

    pltpu.sync_copy(zeros_hbm, wbuf)

    def zinit(i, carry):
        j = s + NS * i

        @pl.when(j < NWB2_I)
        def _():
            pltpu.sync_copy(wbuf, acc.at[pl.ds(j * WB2, WB2)])

        return carry

    lax.fori_loop(0, WB2_ITERS, zinit, 0)
    plsc.subcore_barrier()

    def _pipeline(g_hbm):
        gh = [
            pltpu.async_copy(g_hbm.at[sbuf.at[k]], rbuf.at[k], gsem)
            for k in range(K)
        ]
        sh = []
        for k in range(K):
            gh[k].wait()
            sh.append(
                pltpu.async_copy(rbuf.at[k], acc.at[ibuf.at[k]], ssem, add=True)
            )
        for h in sh:
            h.wait()

    def body(i, carry):
        b = s + NS * i
        row = b * K

        pltpu.sync_copy(src_hbm.at[pl.ds(row, K)], sbuf)
        pltpu.sync_copy(dst_hbm.at[pl.ds(row, K)], ibuf)

        @pl.when(c == 0)
        def _():
            _pipeline(gL_hbm)

        @pl.when(c == 1)
        def _():
            _pipeline(gR_hbm)

        return carry

    lax.fori_loop(0, SUB_ITERS, body, 0)
    plsc.subcore_barrier()

    def wback(i, carry):
        j = s + NS * i

        @pl.when(j < NWB2_O)
        def _():
            pltpu.sync_copy(acc.at[pl.ds(j * WB2, WB2)], wbuf)

            @pl.when(c == 0)
            def _():
                pltpu.sync_copy(wbuf, outL_hbm.at[pl.ds(j * WB2, WB2)])

            @pl.when(c == 1)
            def _():
                pltpu.sync_copy(wbuf, outR_hbm.at[pl.ds(j * WB2, WB2)])

        return carry

    lax.fori_loop(0, WB2_ITERS, wback, 0)


def _prep_body(x0_ref, x1_ref, degA_ref, degB_ref, se_ref, ce_ref,
               g0L_ref, g0R_ref):
    oh_s = (x0_ref[...] == lax.broadcasted_iota(jnp.int32, (BLK, N_SHAPE), 1))
    oh_c = (x1_ref[...] == lax.broadcasted_iota(jnp.int32, (BLK, N_COLOR), 1))
    h0 = jnp.dot(oh_s.astype(jnp.float32), se_ref[...], precision=_PREC)
    h0 = h0 + jnp.dot(oh_c.astype(jnp.float32), ce_ref[...], precision=_PREC)
    dinv = lax.rsqrt(degA_ref[...] + degB_ref[...] + 1.0)
    g = h0 * dinv
    g0L_ref[...] = g[:, :FH]
    g0R_ref[...] = g[:, FH:]


def _layer_body(sL_ref, sR_ref, gL_ref, gR_ref, degA_ref, degB_ref,
                w_ref, b_ref, goutL_ref, goutR_ref):
    dinv = lax.rsqrt(degA_ref[...] + degB_ref[...] + 1.0)
    sg = jnp.concatenate(
        [sL_ref[...] + gL_ref[...], sR_ref[...] + gR_ref[...]], axis=1)
    z = sg * dinv
    h = jnp.maximum(jnp.dot(z, w_ref[...], precision=_PREC) + b_ref[...], 0.0)
    g = h * dinv
    goutL_ref[...] = g[:, :FH]
    goutR_ref[...] = g[:, FH:]


def _pool_body(sL_ref, sR_ref, gL_ref, gR_ref, degA_ref, degB_ref,
               w_ref, b_ref, batch_ref, sums_ref, cnt_ref):
    i = pl.program_id(0)
    dinv = lax.rsqrt(degA_ref[...] + degB_ref[...] + 1.0)
    sg = jnp.concatenate(
        [sL_ref[...] + gL_ref[...], sR_ref[...] + gR_ref[...]], axis=1)
    z = sg * dinv
    h = jnp.maximum(jnp.dot(z, w_ref[...], precision=_PREC) + b_ref[...], 0.0)
    oh = (batch_ref[...] == lax.broadcasted_iota(jnp.int32, (BLK, N_GRAPHS), 1))
    oh = oh.astype(jnp.float32)
    ps = lax.dot_general(oh, h, (((0,), (0,)), ((), ())), precision=_PREC)
    pc = jnp.sum(oh, axis=0)[:, None]

    @pl.when(i == 0)
    def _():
        sums_ref[...] = ps
        cnt_ref[...] = pc

    @pl.when(i != 0)
    def _():
        sums_ref[...] += ps
        cnt_ref[...] += pc


def _head_body(sums_ref, cnt_ref, wl_ref, bl_ref, out_ref):
    hg = sums_ref[...] / jnp.maximum(cnt_ref[...], 1.0)
    out_ref[...] = jnp.dot(hg, wl_ref[...], precision=_PREC) + bl_ref[...]


def _row_spec(width):
    return pl.BlockSpec((BLK, width), lambda i: (i, 0))


def _full_spec(shape):
    return pl.BlockSpec(shape, lambda i: (0, 0))


def kernel(x, edge_index, batch, shape_emb, color_emb, W1, b1, W2, b2, Wlin, blin):
    x0 = x[:, 0:1]
    x1 = x[:, 1:2]
    src = edge_index[0]
    dst = edge_index[1]

    pad = E_PAD - E
    src_p = jnp.concatenate(
        [src.astype(jnp.int32), jnp.zeros((pad,), jnp.int32)]).reshape(NCHP, CHUNK)
    dst_p = jnp.concatenate(
        [dst.astype(jnp.int32),
         N + (jnp.arange(pad, dtype=jnp.int32) % JUNK)]).reshape(NCHP, CHUNK)

    ones128 = jnp.ones((CHUNK,), jnp.float32)
    zeros1d = jnp.zeros((WB,), jnp.float32)
    zeros2d = jnp.zeros((WB2, FH), jnp.float32)

    degA, degB = _deg_sc(dst_p, ones128, zeros1d)
    degA2 = degA[:, None]
    degB2 = degB[:, None]

    g0L, g0R = pl.pallas_call(
        _prep_body,
        grid=(NB,),
        in_specs=[
            _row_spec(1), _row_spec(1), _row_spec(1), _row_spec(1),
            _full_spec((N_SHAPE, F)), _full_spec((N_COLOR, F)),
        ],
        out_specs=[_row_spec(FH), _row_spec(FH)],
        out_shape=[
            jax.ShapeDtypeStruct((N, FH), jnp.float32),
            jax.ShapeDtypeStruct((N, FH), jnp.float32),
        ],
    )(x0, x1, degA2, degB2, shape_emb, color_emb)

    s1L, s1R = _agg_sc(g0L, g0R, src_p, dst_p, zeros2d)

    g1L, g1R = pl.pallas_call(
        _layer_body,
        grid=(NB,),
        in_specs=[
            _row_spec(FH), _row_spec(FH), _row_spec(FH), _row_spec(FH),
            _row_spec(1), _row_spec(1),
            _full_spec((F, F)), _full_spec((1, F)),
        ],
        out_specs=[_row_spec(FH), _row_spec(FH)],
        out_shape=[
            jax.ShapeDtypeStruct((N, FH), jnp.float32),
            jax.ShapeDtypeStruct((N, FH), jnp.float32),
        ],
    )(s1L, s1R, g0L, g0R, degA2, degB2, W1, b1[None, :])

    s2L, s2R = _agg_sc(g1L, g1R, src_p, dst_p, zeros2d)

    sums, cnt = pl.pallas_call(
        _pool_body,
        grid=(NB,),
        in_specs=[
            _row_spec(FH), _row_spec(FH), _row_spec(FH), _row_spec(FH),
            _row_spec(1), _row_spec(1),
            _full_spec((F, F)), _full_spec((1, F)), _row_spec(1),
        ],
        out_specs=[
            _full_spec((N_GRAPHS, F)),
            _full_spec((N_GRAPHS, 1)),
        ],
        out_shape=[
            jax.ShapeDtypeStruct((N_GRAPHS, F), jnp.float32),
            jax.ShapeDtypeStruct((N_GRAPHS, 1), jnp.float32),
        ],
    )(s2L, s2R, g1L, g1R, degA2, degB2, W2, b2[None, :], batch[:, None])

    out = pl.pallas_call(
        _head_body,
        grid=(1,),
        in_specs=[
            _full_spec((N_GRAPHS, F)),
            _full_spec((N_GRAPHS, 1)),
            _full_spec((F, blin.shape[0])),
            _full_spec((1, blin.shape[0])),
        ],
        out_specs=_full_spec((N_GRAPHS, blin.shape[0])),
        out_shape=jax.ShapeDtypeStruct((N_GRAPHS, blin.shape[0]), jnp.float32),
    )(sums, cnt, Wlin, blin[None, :])

    return out


# speedup vs baseline: 35.3084x; 1.3829x over previous
import functools

import jax
import jax.numpy as jnp
from jax import lax
from jax.experimental import pallas as pl
from jax.experimental.pallas import tpu as pltpu
from jax.experimental.pallas import tpu_sc as plsc

N = 100000
E = 1600000
F = 32
FH = F // 2
N_SHAPE = 16
N_COLOR = 8
N_GRAPHS = 256

NC = 2
NS = 16
CHUNK = 128
K = 8
KD = 8
BLK_E = CHUNK * 2 * KD * NS
E_PAD_D = ((E + BLK_E - 1) // BLK_E) * BLK_E
NCHP_D = E_PAD_D // CHUNK
NBLKD_C = NCHP_D // 2 // KD
SUB_ITERS_D = NBLKD_C // NS
EA = E + N
E_PAD_A = ((EA + BLK_E - 1) // BLK_E) * BLK_E
NCHP_A = E_PAD_A // CHUNK
NBLK_A = NCHP_A // K
SUB_ITERS_A = NBLK_A // NS
JUNK = 500
JUNKD = 512
ACC_D = N + JUNKD
ACC_A = N + JUNK
WB = 1000
NWB = N // WB
WB_ITERS = (NWB + NS - 1) // NS
WB2 = 125
NWB2_I = ACC_A // WB2
NWB2_O = N // WB2
WB2_ITERS = (NWB2_I + NS - 1) // NS

_MESH = plsc.VectorSubcoreMesh(
    core_axis_name="c", subcore_axis_name="s", num_cores=NC, num_subcores=NS
)

NP = 102400
MP = NP // 8
PADN = NP - N
BM = 1280
NBM = MP // BM
BLK = 2000
NB = N // BLK

_PREC = lax.Precision.HIGHEST


@functools.partial(
    pl.kernel,
    out_type=[
        jax.ShapeDtypeStruct((N,), jnp.float32),
        jax.ShapeDtypeStruct((N,), jnp.float32),
    ],
    mesh=_MESH,
    compiler_params=pltpu.CompilerParams(use_tc_tiling_on_sc=False),
    scratch_types=[
        pltpu.VMEM_SHARED((ACC_D,), jnp.float32),
        pltpu.VMEM((KD, CHUNK), jnp.int32),
        pltpu.VMEM((CHUNK,), jnp.float32),
        pltpu.VMEM((WB,), jnp.float32),
        pltpu.SemaphoreType.DMA,
    ],
)
def _deg_sc(dst_hbm, ones_hbm, zeros_hbm, degA_hbm, degB_hbm,
            acc, ibuf, obuf, wbuf, sem):
    c = lax.axis_index("c")
    s = lax.axis_index("s")

    pltpu.sync_copy(ones_hbm, obuf)
    pltpu.sync_copy(zeros_hbm, wbuf)

    def zinit(i, carry):
        j = s + NS * i

        @pl.when(j < NWB)
        def _():
            pltpu.sync_copy(wbuf, acc.at[pl.ds(j * WB, WB)])

        return carry

    lax.fori_loop(0, WB_ITERS, zinit, 0)

    @pl.when(s == 0)
    def _():
        pltpu.sync_copy(wbuf.at[pl.ds(0, JUNKD)], acc.at[pl.ds(N, JUNKD)])

    plsc.subcore_barrier()

    def body(i, carry):
        b = c * NBLKD_C + s + NS * i
        row = b * KD
        pltpu.sync_copy(dst_hbm.at[pl.ds(row, KD)], ibuf)
        for k in range(KD):
            pltpu.sync_copy(obuf, acc.at[ibuf.at[k]], add=True)
        return carry

    lax.fori_loop(0, SUB_ITERS_D, body, 0)
    plsc.subcore_barrier()

    def wback(i, carry):
        j = s + NS * i

        @pl.when(j < NWB)
        def _():
            pltpu.sync_copy(acc.at[pl.ds(j * WB, WB)], wbuf)

            @pl.when(c == 0)
            def _():
                pltpu.sync_copy(wbuf, degA_hbm.at[pl.ds(j * WB, WB)])

            @pl.when(c == 1)
            def _():
                pltpu.sync_copy(wbuf, degB_hbm.at[pl.ds(j * WB, WB)])

        return carry

    lax.fori_loop(0, WB_ITERS, wback, 0)


@functools.partial(
    pl.kernel,
    out_type=[
        jax.ShapeDtypeStruct((NP, FH), jnp.float32),
        jax.ShapeDtypeStruct((NP, FH), jnp.float32),
    ],
    mesh=_MESH,
    compiler_params=pltpu.CompilerParams(use_tc_tiling_on_sc=False),
    scratch_types=[
        pltpu.VMEM_SHARED((ACC_A, FH), jnp.float32),
        pltpu.VMEM((K, CHUNK), jnp.int32),
        pltpu.VMEM((K, CHUNK), jnp.int32),
        pltpu.VMEM((K, CHUNK, FH), jnp.float32),
        pltpu.VMEM((WB2, FH), jnp.float32),
        pltpu.SemaphoreType.DMA,
        pltpu.SemaphoreType.DMA,
    ],
)
def _agg_sc(gL_hbm, gR_hbm, src_hbm, dst_hbm, zeros_hbm, outL_hbm, outR_hbm,
            acc, sbuf, ibuf, rbuf, wbuf, gsem, ssem):
    c = lax.axis_index("c")
    s = lax.axis_index("s")

    pltpu.sync_copy(zeros_hbm, wbuf)

    def zinit(i, carry):
        j = s + NS * i

        @pl.when(j < NWB2_I)
        def _():
            pltpu.sync_copy(wbuf, acc.at[pl.ds(j * WB2, WB2)])

        return carry

    lax.fori_loop(0, WB2_ITERS, zinit, 0)
    plsc.subcore_barrier()

    def _pipeline(g_hbm):
        gh = [
            pltpu.async_copy(g_hbm.at[sbuf.at[k]], rbuf.at[k], gsem)
            for k in range(K)
        ]
        sh = []
        for k in range(K):
            gh[k].wait()
            sh.append(
                pltpu.async_copy(rbuf.at[k], acc.at[ibuf.at[k]], ssem, add=True)
            )
        for h in sh:
            h.wait()

    def body(i, carry):
        b = s + NS * i
        row = b * K

        pltpu.sync_copy(src_hbm.at[pl.ds(row, K)], sbuf)
        pltpu.sync_copy(dst_hbm.at[pl.ds(row, K)], ibuf)

        @pl.when(c == 0)
        def _():
            _pipeline(gL_hbm)

        @pl.when(c == 1)
        def _():
            _pipeline(gR_hbm)

        return carry

    lax.fori_loop(0, SUB_ITERS_A, body, 0)
    plsc.subcore_barrier()

    def wback(i, carry):
        j = s + NS * i

        @pl.when(j < NWB2_O)
        def _():
            pltpu.sync_copy(acc.at[pl.ds(j * WB2, WB2)], wbuf)

            @pl.when(c == 0)
            def _():
                pltpu.sync_copy(wbuf, outL_hbm.at[pl.ds(j * WB2, WB2)])

            @pl.when(c == 1)
            def _():
                pltpu.sync_copy(wbuf, outR_hbm.at[pl.ds(j * WB2, WB2)])

        return carry

    lax.fori_loop(0, WB2_ITERS, wback, 0)


def _prep_body(x0_ref, x1_ref, degp_ref, sel_ref, ser_ref, cel_ref, cer_ref,
               g0L_ref, g0R_ref):
    iota16 = lax.broadcasted_iota(jnp.int32, (1, 128), 1) % 16
    ohS = (x0_ref[...] == iota16).astype(jnp.float32)
    ohC = (x1_ref[...] == iota16).astype(jnp.float32)
    hL = (jnp.dot(ohS, sel_ref[...], precision=_PREC)
          + jnp.dot(ohC, cel_ref[...], precision=_PREC))
    hR = (jnp.dot(ohS, ser_ref[...], precision=_PREC)
          + jnp.dot(ohC, cer_ref[...], precision=_PREC))
    dinv = lax.rsqrt(degp_ref[...] + 1.0)
    g0L_ref[...] = hL * dinv
    g0R_ref[...] = hR * dinv


def _layer_body(sL_ref, sR_ref, degp_ref, wll_ref, wrl_ref, wlr_ref, wrr_ref,
                bl_ref, br_ref, gL_ref, gR_ref):
    dinv = lax.rsqrt(degp_ref[...] + 1.0)
    zL = sL_ref[...] * dinv
    zR = sR_ref[...] * dinv
    hL = jnp.maximum(
        jnp.dot(zL, wll_ref[...], precision=_PREC)
        + jnp.dot(zR, wrl_ref[...], precision=_PREC) + bl_ref[...], 0.0)
    hR = jnp.maximum(
        jnp.dot(zL, wlr_ref[...], precision=_PREC)
        + jnp.dot(zR, wrr_ref[...], precision=_PREC) + br_ref[...], 0.0)
    gL_ref[...] = hL * dinv
    gR_ref[...] = hR * dinv


def _pool_body(sL_ref, sR_ref, deg_ref, w_ref, b_ref, batch_ref,
               sums_ref, cnt_ref):
    i = pl.program_id(0)
    dinv = lax.rsqrt(deg_ref[...] + 1.0)
    z = jnp.concatenate([sL_ref[...], sR_ref[...]], axis=1) * dinv
    h = jnp.maximum(jnp.dot(z, w_ref[...], precision=_PREC) + b_ref[...], 0.0)
    oh = (batch_ref[...] == lax.broadcasted_iota(jnp.int32, (BLK, N_GRAPHS), 1))
    oh = oh.astype(jnp.float32)
    ps = lax.dot_general(oh, h, (((0,), (0,)), ((), ())))
    pc = jnp.sum(oh, axis=0)[:, None]

    @pl.when(i == 0)
    def _():
        sums_ref[...] = ps
        cnt_ref[...] = pc

    @pl.when(i != 0)
    def _():
        sums_ref[...] += ps
        cnt_ref[...] += pc


def _head_body(sums_ref, cnt_ref, wl_ref, bl_ref, out_ref):
    hg = sums_ref[...] / jnp.maximum(cnt_ref[...], 1.0)
    out_ref[...] = jnp.dot(hg, wl_ref[...], precision=_PREC) + bl_ref[...]


def _pk_spec():
    return pl.BlockSpec((BM, 128), lambda i: (i, 0))


def _row_spec(width):
    return pl.BlockSpec((BLK, width), lambda i: (i, 0))


def _full_spec(shape):
    return pl.BlockSpec(shape, lambda i: (0, 0))


def _kron8(w):
    return jnp.kron(jnp.eye(8, dtype=jnp.float32), w)


def _pack8(v):
    vp = jnp.concatenate([v, jnp.zeros((PADN,), v.dtype)])
    return jnp.broadcast_to(
        vp.reshape(MP, 8, 1), (MP, 8, 16)).reshape(MP, 128)


def kernel(x, edge_index, batch, shape_emb, color_emb, W1, b1, W2, b2, Wlin, blin):
    src = edge_index[0].astype(jnp.int32)
    dst = edge_index[1].astype(jnp.int32)
    loop = jnp.arange(N, dtype=jnp.int32)

    padD = E_PAD_D - E
    dstD = jnp.concatenate(
        [dst, N + (jnp.arange(padD, dtype=jnp.int32) % JUNK)]).reshape(NCHP_D, CHUNK)
    padA = E_PAD_A - EA
    srcA = jnp.concatenate(
        [src, loop, jnp.zeros((padA,), jnp.int32)]).reshape(NCHP_A, CHUNK)
    dstA = jnp.concatenate(
        [dst, loop,
         N + (jnp.arange(padA, dtype=jnp.int32) % JUNK)]).reshape(NCHP_A, CHUNK)

    ones128 = jnp.ones((CHUNK,), jnp.float32)
    zeros1d = jnp.zeros((WB,), jnp.float32)
    zeros2d = jnp.zeros((WB2, FH), jnp.float32)

    degA, degB = _deg_sc(dstD, ones128, zeros1d)
    deg = degA + degB
    degp = _pack8(deg)
    deg2 = deg[:, None]

    x0p = _pack8(x[:, 0].astype(jnp.int32))
    x1p = _pack8(x[:, 1].astype(jnp.int32))
    ce16 = jnp.zeros((N_SHAPE, F), jnp.float32).at[:N_COLOR].set(color_emb)
    seL = _kron8(shape_emb[:, :FH])
    seR = _kron8(shape_emb[:, FH:])
    ceL = _kron8(ce16[:, :FH])
    ceR = _kron8(ce16[:, FH:])
    w1ll = _kron8(W1[:FH, :FH])
    w1rl = _kron8(W1[FH:, :FH])
    w1lr = _kron8(W1[:FH, FH:])
    w1rr = _kron8(W1[FH:, FH:])
    b1l = jnp.tile(b1[:FH], 8)[None, :]
    b1r = jnp.tile(b1[FH:], 8)[None, :]

    g0Lp, g0Rp = pl.pallas_call(
        _prep_body,
        grid=(NBM,),
        in_specs=[
            _pk_spec(), _pk_spec(), _pk_spec(),
            _full_spec((128, 128)), _full_spec((128, 128)),
            _full_spec((128, 128)), _full_spec((128, 128)),
        ],
        out_specs=[_pk_spec(), _pk_spec()],
        out_shape=[
            jax.ShapeDtypeStruct((MP, 128), jnp.float32),
            jax.ShapeDtypeStruct((MP, 128), jnp.float32),
        ],
    )(x0p, x1p, degp, seL, seR, ceL, ceR)

    s1L, s1R = _agg_sc(
        g0Lp.reshape(NP, FH), g0Rp.reshape(NP, FH), srcA, dstA, zeros2d)

    g1Lp, g1Rp = pl.pallas_call(
        _layer_body,
        grid=(NBM,),
        in_specs=[
            _pk_spec(), _pk_spec(), _pk_spec(),
            _full_spec((128, 128)), _full_spec((128, 128)),
            _full_spec((128, 128)), _full_spec((128, 128)),
            _full_spec((1, 128)), _full_spec((1, 128)),
        ],
        out_specs=[_pk_spec(), _pk_spec()],
        out_shape=[
            jax.ShapeDtypeStruct((MP, 128), jnp.float32),
            jax.ShapeDtypeStruct((MP, 128), jnp.float32),
        ],
    )(s1L.reshape(MP, 128), s1R.reshape(MP, 128), degp,
      w1ll, w1rl, w1lr, w1rr, b1l, b1r)

    s2L, s2R = _agg_sc(
        g1Lp.reshape(NP, FH), g1Rp.reshape(NP, FH), srcA, dstA, zeros2d)

    sums, cnt = pl.pallas_call(
        _pool_body,
        grid=(NB,),
        in_specs=[
            _row_spec(FH), _row_spec(FH), _row_spec(1),
            _full_spec((F, F)), _full_spec((1, F)), _row_spec(1),
        ],
        out_specs=[
            _full_spec((N_GRAPHS, F)),
            _full_spec((N_GRAPHS, 1)),
        ],
        out_shape=[
            jax.ShapeDtypeStruct((N_GRAPHS, F), jnp.float32),
            jax.ShapeDtypeStruct((N_GRAPHS, 1), jnp.float32),
        ],
    )(s2L, s2R, deg2, W2, b2[None, :], batch[:, None])

    out = pl.pallas_call(
        _head_body,
        grid=(1,),
        in_specs=[
            _full_spec((N_GRAPHS, F)),
            _full_spec((N_GRAPHS, 1)),
            _full_spec((F, blin.shape[0])),
            _full_spec((1, blin.shape[0])),
        ],
        out_specs=_full_spec((N_GRAPHS, blin.shape[0])),
        out_shape=jax.ShapeDtypeStruct((N_GRAPHS, blin.shape[0]), jnp.float32),
    )(sums, cnt, Wlin, blin[None, :])

    return out


# speedup vs baseline: 35.3352x; 1.0008x over previous
import functools

import jax
import jax.numpy as jnp
from jax import lax
from jax.experimental import pallas as pl
from jax.experimental.pallas import tpu as pltpu
from jax.experimental.pallas import tpu_sc as plsc

N = 100000
E = 1600000
F = 32
FH = F // 2
N_SHAPE = 16
N_COLOR = 8
N_GRAPHS = 256

NC = 2
NS = 16
CHUNK = 128
K = 8
KD = 8
BLK_E = CHUNK * 2 * KD * NS
E_PAD_D = ((E + BLK_E - 1) // BLK_E) * BLK_E
NCHP_D = E_PAD_D // CHUNK
NBLKD_C = NCHP_D // 2 // KD
SUB_ITERS_D = NBLKD_C // NS
EA = E + N
E_PAD_A = ((EA + BLK_E - 1) // BLK_E) * BLK_E
NCHP_A = E_PAD_A // CHUNK
NBLK_A = NCHP_A // K
SUB_ITERS_A = NBLK_A // NS
JUNK = 500
JUNKD = 512
ACC_D = N + JUNKD
ACC_A = N + JUNK
WB = 1000
NWB = N // WB
WB_ITERS = (NWB + NS - 1) // NS
WB2 = 125
NWB2_I = ACC_A // WB2
NWB2_O = N // WB2
WB2_ITERS = (NWB2_I + NS - 1) // NS

_MESH = plsc.VectorSubcoreMesh(
    core_axis_name="c", subcore_axis_name="s", num_cores=NC, num_subcores=NS
)

NP = 102400
MP = NP // 8
PADN = NP - N
BM = 1280
NBM = MP // BM
BLK = 2000
NB = N // BLK

_PREC = lax.Precision.HIGHEST


@functools.partial(
    pl.kernel,
    out_type=[
        jax.ShapeDtypeStruct((N,), jnp.float32),
        jax.ShapeDtypeStruct((N,), jnp.float32),
    ],
    mesh=_MESH,
    compiler_params=pltpu.CompilerParams(use_tc_tiling_on_sc=False),
    scratch_types=[
        pltpu.VMEM_SHARED((ACC_D,), jnp.float32),
        pltpu.VMEM((KD, CHUNK), jnp.int32),
        pltpu.VMEM((CHUNK,), jnp.float32),
        pltpu.VMEM((WB,), jnp.float32),
        pltpu.SemaphoreType.DMA,
    ],
)
def _deg_sc(dst_hbm, ones_hbm, zeros_hbm, degA_hbm, degB_hbm,
            acc, ibuf, obuf, wbuf, sem):
    c = lax.axis_index("c")
    s = lax.axis_index("s")

    pltpu.sync_copy(ones_hbm, obuf)
    pltpu.sync_copy(zeros_hbm, wbuf)

    def zinit(i, carry):
        j = s + NS * i

        @pl.when(j < NWB)
        def _():
            pltpu.sync_copy(wbuf, acc.at[pl.ds(j * WB, WB)])

        return carry

    lax.fori_loop(0, WB_ITERS, zinit, 0)

    @pl.when(s == 0)
    def _():
        pltpu.sync_copy(wbuf.at[pl.ds(0, JUNKD)], acc.at[pl.ds(N, JUNKD)])

    plsc.subcore_barrier()

    def body(i, carry):
        b = c * NBLKD_C + s + NS * i
        row = b * KD
        pltpu.sync_copy(dst_hbm.at[pl.ds(row, KD)], ibuf)
        hs = [
            pltpu.async_copy(obuf, acc.at[ibuf.at[k]], sem, add=True)
            for k in range(KD)
        ]
        for h in hs:
            h.wait()
        return carry

    lax.fori_loop(0, SUB_ITERS_D, body, 0)
    plsc.subcore_barrier()

    def wback(i, carry):
        j = s + NS * i

        @pl.when(j < NWB)
        def _():
            pltpu.sync_copy(acc.at[pl.ds(j * WB, WB)], wbuf)

            @pl.when(c == 0)
            def _():
                pltpu.sync_copy(wbuf, degA_hbm.at[pl.ds(j * WB, WB)])

            @pl.when(c == 1)
            def _():
                pltpu.sync_copy(wbuf, degB_hbm.at[pl.ds(j * WB, WB)])

        return carry

    lax.fori_loop(0, WB_ITERS, wback, 0)


@functools.partial(
    pl.kernel,
    out_type=[
        jax.ShapeDtypeStruct((NP, FH), jnp.float32),
        jax.ShapeDtypeStruct((NP, FH), jnp.float32),
    ],
    mesh=_MESH,
    compiler_params=pltpu.CompilerParams(use_tc_tiling_on_sc=False),
    scratch_types=[
        pltpu.VMEM_SHARED((ACC_A, FH), jnp.float32),
        pltpu.VMEM((K, CHUNK), jnp.int32),
        pltpu.VMEM((K, CHUNK), jnp.int32),
        pltpu.VMEM((K, CHUNK, FH), jnp.float32),
        pltpu.VMEM((WB2, FH), jnp.float32),
        pltpu.SemaphoreType.DMA,
        pltpu.SemaphoreType.DMA,
    ],
)
def _agg_sc(gL_hbm, gR_hbm, src_hbm, dst_hbm, zeros_hbm, outL_hbm, outR_hbm,
            acc, sbuf, ibuf, rbuf, wbuf, gsem, ssem):
    c = lax.axis_index("c")
    s = lax.axis_index("s")

    pltpu.sync_copy(zeros_hbm, wbuf)

    def zinit(i, carry):
        j = s + NS * i

        @pl.when(j < NWB2_I)
        def _():
            pltpu.sync_copy(wbuf, acc.at[pl.ds(j * WB2, WB2)])

        return carry

    lax.fori_loop(0, WB2_ITERS, zinit, 0)
    plsc.subcore_barrier()

    def _pipeline(g_hbm):
        gh = [
            pltpu.async_copy(g_hbm.at[sbuf.at[k]], rbuf.at[k], gsem)
            for k in range(K)
        ]
        sh = []
        for k in range(K):
            gh[k].wait()
            sh.append(
                pltpu.async_copy(rbuf.at[k], acc.at[ibuf.at[k]], ssem, add=True)
            )
        for h in sh:
            h.wait()

    def body(i, carry):
        b = s + NS * i
        row = b * K

        pltpu.sync_copy(src_hbm.at[pl.ds(row, K)], sbuf)
        pltpu.sync_copy(dst_hbm.at[pl.ds(row, K)], ibuf)

        @pl.when(c == 0)
        def _():
            _pipeline(gL_hbm)

        @pl.when(c == 1)
        def _():
            _pipeline(gR_hbm)

        return carry

    lax.fori_loop(0, SUB_ITERS_A, body, 0)
    plsc.subcore_barrier()

    def wback(i, carry):
        j = s + NS * i

        @pl.when(j < NWB2_O)
        def _():
            pltpu.sync_copy(acc.at[pl.ds(j * WB2, WB2)], wbuf)

            @pl.when(c == 0)
            def _():
                pltpu.sync_copy(wbuf, outL_hbm.at[pl.ds(j * WB2, WB2)])

            @pl.when(c == 1)
            def _():
                pltpu.sync_copy(wbuf, outR_hbm.at[pl.ds(j * WB2, WB2)])

        return carry

    lax.fori_loop(0, WB2_ITERS, wback, 0)


def _prep_body(x0_ref, x1_ref, degp_ref, sel_ref, ser_ref, cel_ref, cer_ref,
               g0L_ref, g0R_ref):
    iota16 = lax.broadcasted_iota(jnp.int32, (1, 128), 1) % 16
    ohS = (x0_ref[...] == iota16).astype(jnp.float32)
    ohC = (x1_ref[...] == iota16).astype(jnp.float32)
    hL = (jnp.dot(ohS, sel_ref[...], precision=_PREC)
          + jnp.dot(ohC, cel_ref[...], precision=_PREC))
    hR = (jnp.dot(ohS, ser_ref[...], precision=_PREC)
          + jnp.dot(ohC, cer_ref[...], precision=_PREC))
    dinv = lax.rsqrt(degp_ref[...] + 1.0)
    g0L_ref[...] = hL * dinv
    g0R_ref[...] = hR * dinv


def _layer_body(sL_ref, sR_ref, degp_ref, wll_ref, wrl_ref, wlr_ref, wrr_ref,
                bl_ref, br_ref, gL_ref, gR_ref):
    dinv = lax.rsqrt(degp_ref[...] + 1.0)
    zL = sL_ref[...] * dinv
    zR = sR_ref[...] * dinv
    hL = jnp.maximum(
        jnp.dot(zL, wll_ref[...], precision=_PREC)
        + jnp.dot(zR, wrl_ref[...], precision=_PREC) + bl_ref[...], 0.0)
    hR = jnp.maximum(
        jnp.dot(zL, wlr_ref[...], precision=_PREC)
        + jnp.dot(zR, wrr_ref[...], precision=_PREC) + br_ref[...], 0.0)
    gL_ref[...] = hL * dinv
    gR_ref[...] = hR * dinv


def _pool_body(sL_ref, sR_ref, deg_ref, w_ref, b_ref, batch_ref,
               sums_ref, cnt_ref):
    i = pl.program_id(0)
    dinv = lax.rsqrt(deg_ref[...] + 1.0)
    z = jnp.concatenate([sL_ref[...], sR_ref[...]], axis=1) * dinv
    h = jnp.maximum(jnp.dot(z, w_ref[...], precision=_PREC) + b_ref[...], 0.0)
    oh = (batch_ref[...] == lax.broadcasted_iota(jnp.int32, (BLK, N_GRAPHS), 1))
    oh = oh.astype(jnp.float32)
    ps = lax.dot_general(oh, h, (((0,), (0,)), ((), ())))
    pc = jnp.sum(oh, axis=0)[:, None]

    @pl.when(i == 0)
    def _():
        sums_ref[...] = ps
        cnt_ref[...] = pc

    @pl.when(i != 0)
    def _():
        sums_ref[...] += ps
        cnt_ref[...] += pc


def _head_body(sums_ref, cnt_ref, wl_ref, bl_ref, out_ref):
    hg = sums_ref[...] / jnp.maximum(cnt_ref[...], 1.0)
    out_ref[...] = jnp.dot(hg, wl_ref[...], precision=_PREC) + bl_ref[...]


def _pk_spec():
    return pl.BlockSpec((BM, 128), lambda i: (i, 0))


def _row_spec(width):
    return pl.BlockSpec((BLK, width), lambda i: (i, 0))


def _full_spec(shape):
    return pl.BlockSpec(shape, lambda i: (0, 0))


def _kron8(w):
    return jnp.kron(jnp.eye(8, dtype=jnp.float32), w)


def _pack8(v):
    vp = jnp.concatenate([v, jnp.zeros((PADN,), v.dtype)])
    return jnp.broadcast_to(
        vp.reshape(MP, 8, 1), (MP, 8, 16)).reshape(MP, 128)


def kernel(x, edge_index, batch, shape_emb, color_emb, W1, b1, W2, b2, Wlin, blin):
    src = edge_index[0].astype(jnp.int32)
    dst = edge_index[1].astype(jnp.int32)
    loop = jnp.arange(N, dtype=jnp.int32)

    padD = E_PAD_D - E
    dstD = jnp.concatenate(
        [dst, N + (jnp.arange(padD, dtype=jnp.int32) % JUNK)]).reshape(NCHP_D, CHUNK)
    padA = E_PAD_A - EA
    srcA = jnp.concatenate(
        [src, loop, jnp.zeros((padA,), jnp.int32)]).reshape(NCHP_A, CHUNK)
    dstA = jnp.concatenate(
        [dst, loop,
         N + (jnp.arange(padA, dtype=jnp.int32) % JUNK)]).reshape(NCHP_A, CHUNK)

    ones128 = jnp.ones((CHUNK,), jnp.float32)
    zeros1d = jnp.zeros((WB,), jnp.float32)
    zeros2d = jnp.zeros((WB2, FH), jnp.float32)

    degA, degB = _deg_sc(dstD, ones128, zeros1d)
    deg = degA + degB
    degp = _pack8(deg)
    deg2 = deg[:, None]

    x0p = _pack8(x[:, 0].astype(jnp.int32))
    x1p = _pack8(x[:, 1].astype(jnp.int32))
    ce16 = jnp.zeros((N_SHAPE, F), jnp.float32).at[:N_COLOR].set(color_emb)
    seL = _kron8(shape_emb[:, :FH])
    seR = _kron8(shape_emb[:, FH:])
    ceL = _kron8(ce16[:, :FH])
    ceR = _kron8(ce16[:, FH:])
    w1ll = _kron8(W1[:FH, :FH])
    w1rl = _kron8(W1[FH:, :FH])
    w1lr = _kron8(W1[:FH, FH:])
    w1rr = _kron8(W1[FH:, FH:])
    b1l = jnp.tile(b1[:FH], 8)[None, :]
    b1r = jnp.tile(b1[FH:], 8)[None, :]

    g0Lp, g0Rp = pl.pallas_call(
        _prep_body,
        grid=(NBM,),
        in_specs=[
            _pk_spec(), _pk_spec(), _pk_spec(),
            _full_spec((128, 128)), _full_spec((128, 128)),
            _full_spec((128, 128)), _full_spec((128, 128)),
        ],
        out_specs=[_pk_spec(), _pk_spec()],
        out_shape=[
            jax.ShapeDtypeStruct((MP, 128), jnp.float32),
            jax.ShapeDtypeStruct((MP, 128), jnp.float32),
        ],
    )(x0p, x1p, degp, seL, seR, ceL, ceR)

    s1L, s1R = _agg_sc(
        g0Lp.reshape(NP, FH), g0Rp.reshape(NP, FH), srcA, dstA, zeros2d)

    g1Lp, g1Rp = pl.pallas_call(
        _layer_body,
        grid=(NBM,),
        in_specs=[
            _pk_spec(), _pk_spec(), _pk_spec(),
            _full_spec((128, 128)), _full_spec((128, 128)),
            _full_spec((128, 128)), _full_spec((128, 128)),
            _full_spec((1, 128)), _full_spec((1, 128)),
        ],
        out_specs=[_pk_spec(), _pk_spec()],
        out_shape=[
            jax.ShapeDtypeStruct((MP, 128), jnp.float32),
            jax.ShapeDtypeStruct((MP, 128), jnp.float32),
        ],
    )(s1L.reshape(MP, 128), s1R.reshape(MP, 128), degp,
      w1ll, w1rl, w1lr, w1rr, b1l, b1r)

    s2L, s2R = _agg_sc(
        g1Lp.reshape(NP, FH), g1Rp.reshape(NP, FH), srcA, dstA, zeros2d)

    sums, cnt = pl.pallas_call(
        _pool_body,
        grid=(NB,),
        in_specs=[
            _row_spec(FH), _row_spec(FH), _row_spec(1),
            _full_spec((F, F)), _full_spec((1, F)), _row_spec(1),
        ],
        out_specs=[
            _full_spec((N_GRAPHS, F)),
            _full_spec((N_GRAPHS, 1)),
        ],
        out_shape=[
            jax.ShapeDtypeStruct((N_GRAPHS, F), jnp.float32),
            jax.ShapeDtypeStruct((N_GRAPHS, 1), jnp.float32),
        ],
    )(s2L, s2R, deg2, W2, b2[None, :], batch[:, None])

    out = pl.pallas_call(
        _head_body,
        grid=(1,),
        in_specs=[
            _full_spec((N_GRAPHS, F)),
            _full_spec((N_GRAPHS, 1)),
            _full_spec((F, blin.shape[0])),
            _full_spec((1, blin.shape[0])),
        ],
        out_specs=_full_spec((N_GRAPHS, blin.shape[0])),
        out_shape=jax.ShapeDtypeStruct((N_GRAPHS, blin.shape[0]), jnp.float32),
    )(sums, cnt, Wlin, blin[None, :])

    return out
